# baseline (speedup 1.0000x reference)
import jax
import jax.numpy as jnp
from jax import lax
from jax.experimental import pallas as pl
from jax.experimental.pallas import tpu as pltpu

_BATCH = 8
_SEQ = 50
_VOCAB = 100000
_ALPHA = 1.0


def _body(chosen_smem, chosen_v, mask_v, rew_v, probs_hbm, out_v, gath_v, sem):
    copies = []
    for b in range(_BATCH):
        for t in range(_SEQ):
            v = chosen_smem[b, t]
            start = pl.multiple_of(v & ~127, 128)
            c = pltpu.make_async_copy(
                probs_hbm.at[t, b, pl.ds(start, 128)],
                gath_v.at[b, t],
                sem,
            )
            c.start()
            copies.append(c)
    pltpu.make_async_copy(gath_v, gath_v, sem).wait()

    tok = chosen_v[...]
    lanesel = (tok & 127)[..., None]
    lane = lax.broadcasted_iota(jnp.int32, (_BATCH, _SEQ, 128), 2)
    p = jnp.sum(jnp.where(lane == lanesel, gath_v[...], 0.0), axis=2)
    m = mask_v[...]
    loss = -jnp.log(p) * m
    s = jnp.sum(loss, axis=1)
    n = jnp.sum(m, axis=1)
    out_v[...] = s * rew_v[...] / n * _ALPHA


def kernel(chosen_tokens, probs, time_step_mask, delta_rewards):
    probs_t = jnp.transpose(probs, (1, 0, 2))
    return pl.pallas_call(
        _body,
        out_shape=jax.ShapeDtypeStruct((_BATCH,), jnp.float32),
        in_specs=[
            pl.BlockSpec(memory_space=pltpu.SMEM),
            pl.BlockSpec(memory_space=pltpu.VMEM),
            pl.BlockSpec(memory_space=pltpu.VMEM),
            pl.BlockSpec(memory_space=pltpu.VMEM),
            pl.BlockSpec(memory_space=pl.ANY),
        ],
        out_specs=pl.BlockSpec(memory_space=pltpu.VMEM),
        scratch_shapes=[
            pltpu.VMEM((_BATCH, _SEQ, 128), jnp.float32),
            pltpu.SemaphoreType.DMA,
        ],
    )(chosen_tokens, chosen_tokens, time_step_mask, delta_rewards, probs_t)


# speedup vs baseline: 1.0027x; 1.0027x over previous
import jax
import jax.numpy as jnp
from jax import lax
from jax.experimental import pallas as pl
from jax.experimental.pallas import tpu as pltpu

_BATCH = 8
_SEQ = 50
_VOCAB = 100000
_ALPHA = 1.0


def _body(chosen_smem, chosen_v, mask_v, rew_v, probs_hbm, out_v, gath_v, sem):
    copies = []
    for t in range(_SEQ):
        for b in range(_BATCH):
            v = chosen_smem[b, t]
            start = pl.multiple_of(v & ~127, 128)
            c = pltpu.make_async_copy(
                probs_hbm.at[t, b, pl.ds(start, 128)],
                gath_v.at[b, t],
                sem,
            )
            c.start()
            copies.append(c)
    pltpu.make_async_copy(gath_v, gath_v, sem).wait()

    tok = chosen_v[...]
    lanesel = (tok & 127)[..., None]
    lane = lax.broadcasted_iota(jnp.int32, (_BATCH, _SEQ, 128), 2)
    p = jnp.sum(jnp.where(lane == lanesel, gath_v[...], 0.0), axis=2)
    m = mask_v[...]
    loss = -jnp.log(p) * m
    s = jnp.sum(loss, axis=1)
    n = jnp.sum(m, axis=1)
    out_v[...] = s * rew_v[...] / n * _ALPHA


def kernel(chosen_tokens, probs, time_step_mask, delta_rewards):
    probs_t = jnp.transpose(probs, (1, 0, 2))
    return pl.pallas_call(
        _body,
        out_shape=jax.ShapeDtypeStruct((_BATCH,), jnp.float32),
        in_specs=[
            pl.BlockSpec(memory_space=pltpu.SMEM),
            pl.BlockSpec(memory_space=pltpu.VMEM),
            pl.BlockSpec(memory_space=pltpu.VMEM),
            pl.BlockSpec(memory_space=pltpu.VMEM),
            pl.BlockSpec(memory_space=pl.ANY),
        ],
        out_specs=pl.BlockSpec(memory_space=pltpu.VMEM),
        scratch_shapes=[
            pltpu.VMEM((_BATCH, _SEQ, 128), jnp.float32),
            pltpu.SemaphoreType.DMA,
        ],
    )(chosen_tokens, chosen_tokens, time_step_mask, delta_rewards, probs_t)
